# baseline (speedup 1.0000x reference)
import functools

import jax
import jax.numpy as jnp
from jax import lax
from jax.experimental import pallas as pl
from jax.experimental.pallas import tpu as pltpu
from jax.experimental.pallas import tpu_sc as plsc

N = 10000
E = 320000
NT = 10240
R = 640
GRID = NT // R
NC = 2
NS = 16
NW = NC * NS
K = 128
EP = 327680
RING = 4
ROWS_PER_TILE = NT // NS


def _dense1_body(x_ref, wl_ref, wr_ref, b1_ref, xl_ref, xr_ref):
    x = x_ref[...]
    xl_ref[...] = jnp.dot(x, wl_ref[...], preferred_element_type=jnp.float32)
    xr_ref[...] = (jnp.dot(x, wr_ref[...], preferred_element_type=jnp.float32)
                   + b1_ref[...])


def _dense1(x_pad, W1l_T, W1r_T, b1):
    return pl.pallas_call(
        _dense1_body,
        grid=(GRID,),
        in_specs=[
            pl.BlockSpec((R, 128), lambda i: (i, 0)),
            pl.BlockSpec((128, 128), lambda i: (0, 0)),
            pl.BlockSpec((128, 128), lambda i: (0, 0)),
            pl.BlockSpec((1, 128), lambda i: (0, 0)),
        ],
        out_specs=[
            pl.BlockSpec((R, 128), lambda i: (i, 0)),
            pl.BlockSpec((R, 128), lambda i: (i, 0)),
        ],
        out_shape=[
            jax.ShapeDtypeStruct((NT, 128), jnp.float32),
            jax.ShapeDtypeStruct((NT, 128), jnp.float32),
        ],
    )(x_pad, W1l_T, W1r_T, b1)


def _dense2_body(agg_ref, cnt_ref, xr_ref, w2l_ref, w2r_ref, b2_ref,
                 hl_ref, hr_ref):
    agg = jnp.concatenate([agg_ref[0], agg_ref[1]], axis=1)
    cnt = 0.5 * (cnt_ref[0] + cnt_ref[1])
    inv = 1.0 / jnp.maximum(cnt, 1.0)
    h1 = jnp.maximum(agg * inv[:, None] + xr_ref[...], 0.0)
    hl_ref[...] = jnp.dot(h1, w2l_ref[...], preferred_element_type=jnp.float32)
    hr_ref[...] = (jnp.dot(h1, w2r_ref[...], preferred_element_type=jnp.float32)
                   + b2_ref[...])


def _dense2(aggP, cntP, xr, W2l_T, W2r_T, b2):
    return pl.pallas_call(
        _dense2_body,
        grid=(GRID,),
        in_specs=[
            pl.BlockSpec((2, R, 64), lambda i: (0, i, 0)),
            pl.BlockSpec((2, R), lambda i: (0, i)),
            pl.BlockSpec((R, 128), lambda i: (i, 0)),
            pl.BlockSpec((128, 64), lambda i: (0, 0)),
            pl.BlockSpec((128, 64), lambda i: (0, 0)),
            pl.BlockSpec((1, 64), lambda i: (0, 0)),
        ],
        out_specs=[
            pl.BlockSpec((R, 64), lambda i: (i, 0)),
            pl.BlockSpec((R, 64), lambda i: (i, 0)),
        ],
        out_shape=[
            jax.ShapeDtypeStruct((NT, 64), jnp.float32),
            jax.ShapeDtypeStruct((NT, 64), jnp.float32),
        ],
    )(aggP, cntP, xr, W2l_T, W2r_T, b2)


def _dense3_body(agg_ref, cnt_ref, hr_ref, wfc_ref, bfc_ref, out_ref):
    agg = agg_ref[0] + agg_ref[1]
    cnt = 0.5 * (cnt_ref[0] + cnt_ref[1])
    inv = 1.0 / jnp.maximum(cnt, 1.0)
    h2 = jnp.maximum(agg * inv[:, None] + hr_ref[...], 0.0)
    logit = jnp.sum(h2 * wfc_ref[...], axis=1, keepdims=True) + bfc_ref[...]
    out_ref[...] = jax.nn.sigmoid(logit)


def _dense3(agg2P, cntP, hr, Wfc, bfc):
    return pl.pallas_call(
        _dense3_body,
        grid=(GRID,),
        in_specs=[
            pl.BlockSpec((2, R, 64), lambda i: (0, i, 0)),
            pl.BlockSpec((2, R), lambda i: (0, i)),
            pl.BlockSpec((R, 64), lambda i: (i, 0)),
            pl.BlockSpec((1, 64), lambda i: (0, 0)),
            pl.BlockSpec((1, 1), lambda i: (0, 0)),
        ],
        out_specs=pl.BlockSpec((R, 1), lambda i: (i, 0)),
        out_shape=jax.ShapeDtypeStruct((NT, 1), jnp.float32),
    )(agg2P, cntP, hr, Wfc, bfc)


def _make_agg(feature_split, with_cnt):
    n_slabs = NS if feature_split else NW
    nb = EP // (n_slabs * K)
    ngrp = nb // RING
    table_rows = 2 * NT if feature_split else NT

    mesh = plsc.VectorSubcoreMesh(
        core_axis_name="c", subcore_axis_name="s",
        num_cores=NC, num_subcores=NS)

    out_type = [jax.ShapeDtypeStruct((NC, NT, 64), jnp.float32)]
    if with_cnt:
        out_type.append(jax.ShapeDtypeStruct((NC, NT), jnp.float32))

    scratch = [
        pltpu.VMEM((nb, K), jnp.int32),
        pltpu.VMEM((nb, K), jnp.int32),
        [pltpu.VMEM((K, 64), jnp.float32) for _ in range(RING)],
        pltpu.VMEM((K,), jnp.float32),
        pltpu.VMEM_SHARED((NT, 64), jnp.float32),
        pltpu.VMEM_SHARED((NT,), jnp.float32),
        [pltpu.SemaphoreType.DMA for _ in range(RING)],
        [pltpu.SemaphoreType.DMA for _ in range(RING)],
        [pltpu.SemaphoreType.DMA for _ in range(RING)],
        pltpu.SemaphoreType.DMA,
    ]

    @functools.partial(pl.kernel, out_type=out_type, mesh=mesh,
                       scratch_types=scratch,
                       compiler_params=pltpu.CompilerParams(
                           use_tc_tiling_on_sc=False))
    def agg_kernel(table, src, dst, zrows, zcnt, *refs):
        if with_cnt:
            out, cnt_out = refs[0], refs[1]
            rest = refs[2:]
        else:
            out = refs[0]
            rest = refs[1:]
        (srcs, dsts, rows, ones, acc, cacc, gsem, ssem, csem, sem) = rest

        cid = lax.axis_index("c")
        sid = lax.axis_index("s")
        slab = sid if feature_split else cid * NS + sid

        zbase = sid * ROWS_PER_TILE
        pltpu.async_copy(src.at[slab], srcs, sem)
        pltpu.sync_copy(dst.at[slab], dsts)
        pltpu.sync_copy(zrows, acc.at[pl.ds(zbase, ROWS_PER_TILE)])
        if with_cnt:
            pltpu.sync_copy(zcnt, cacc.at[pl.ds(zbase, ROWS_PER_TILE)])
            for j in range(K // 16):
                ones[pl.ds(16 * j, 16)] = jnp.full((16,), 1.0, jnp.float32)
        pltpu.make_async_copy(src.at[slab], srcs, sem).wait()
        if feature_split:
            flat_iters = nb * K // 16

            def remap_body(i, carry):
                r = i // (K // 16)
                c = (i % (K // 16)) * 16
                v = srcs[r, pl.ds(c, 16)]
                srcs[r, pl.ds(c, 16)] = v * 2 + cid
                return carry
            lax.fori_loop(0, flat_iters, remap_body, 0)
        plsc.subcore_barrier()

        def wait_gather(b):
            pltpu.make_async_copy(table.at[srcs.at[0]], rows[b],
                                  gsem[b]).wait()

        def wait_scatter(b):
            pltpu.make_async_copy(rows[b], acc.at[dsts.at[0]],
                                  ssem[b]).wait()
            if with_cnt:
                pltpu.make_async_copy(ones, cacc.at[dsts.at[0]],
                                      csem[b]).wait()

        def group(io, first):
            for b in range(RING):
                ib = io * RING + b
                if not first:
                    wait_scatter(b)
                pltpu.async_copy(table.at[srcs.at[ib]], rows[b], gsem[b])
            for b in range(RING):
                ib = io * RING + b
                wait_gather(b)
                pltpu.async_copy(rows[b], acc.at[dsts.at[ib]], ssem[b],
                                 add=True)
                if with_cnt:
                    pltpu.async_copy(ones, cacc.at[dsts.at[ib]], csem[b],
                                     add=True)

        group(0, True)

        def body(io, carry):
            group(io, False)
            return carry

        lax.fori_loop(1, ngrp, body, 0)
        for b in range(RING):
            wait_scatter(b)
        plsc.subcore_barrier()

        pltpu.sync_copy(acc.at[pl.ds(zbase, ROWS_PER_TILE)],
                        out.at[cid, pl.ds(zbase, ROWS_PER_TILE)])
        if with_cnt:
            pltpu.sync_copy(cacc.at[pl.ds(zbase, ROWS_PER_TILE)],
                            cnt_out.at[cid, pl.ds(zbase, ROWS_PER_TILE)])

    return agg_kernel


_agg128 = _make_agg(feature_split=True, with_cnt=True)
_agg64 = _make_agg(feature_split=False, with_cnt=False)


def kernel(x, edge_index, W1l, b1, W1r, W2l, b2, W2r, Wfc, bfc):
    x_pad = jnp.pad(x, ((0, NT - N), (0, 0)))
    src_flat = jnp.pad(edge_index[0], (0, EP - E))
    dst_flat = jnp.pad(edge_index[1], (0, EP - E), constant_values=NT - 1)
    src16 = src_flat.reshape(NS, EP // (NS * K), K)
    dst16 = dst_flat.reshape(NS, EP // (NS * K), K)
    src32 = src_flat.reshape(NW, EP // (NW * K), K)
    dst32 = dst_flat.reshape(NW, EP // (NW * K), K)
    zrows64 = jnp.zeros((ROWS_PER_TILE, 64), jnp.float32)
    zcnt = jnp.zeros((ROWS_PER_TILE,), jnp.float32)

    xl, xr = _dense1(x_pad, W1l.T, W1r.T, b1.reshape(1, 128))
    aggP, cntP = _agg128(xl.reshape(2 * NT, 64), src16, dst16, zrows64, zcnt)
    hl, hr = _dense2(aggP, cntP, xr, W2l.T, W2r.T, b2.reshape(1, 64))
    (agg2P,) = _agg64(hl, src32, dst32, zrows64, zcnt)
    out = _dense3(agg2P, cntP, hr, Wfc.reshape(1, 64), bfc.reshape(1, 1))
    return out[:N, 0]


# speedup vs baseline: 1.2035x; 1.2035x over previous
import functools

import jax
import jax.numpy as jnp
import numpy as np
from jax import lax
from jax.experimental import pallas as pl
from jax.experimental.pallas import tpu as pltpu
from jax.experimental.pallas import tpu_sc as plsc

N = 10000
E = 320000
NT = 10240
R = 640
GRID = NT // R
NC = 2
NS = 16
NW = NC * NS
K = 128
EP = 327680
RING = 4
ROWS_PER_TILE = NT // NS

_G = np.array([2 * p for p in range(16)] + [2 * p + 1 for p in range(16)]
              + [32 + 2 * p for p in range(16)]
              + [33 + 2 * p for p in range(16)])
_H = np.argsort(_G)


def _dense1_body(x_ref, wl_ref, wr_ref, b1_ref, xlb_ref, xr_ref):
    x = x_ref[...]
    xl = jnp.dot(x, wl_ref[...], preferred_element_type=jnp.float32)
    xlb = xl.astype(jnp.bfloat16)
    xlb_ref[0] = xlb[:, :64]
    xlb_ref[1] = xlb[:, 64:]
    xr_ref[...] = (jnp.dot(x, wr_ref[...], preferred_element_type=jnp.float32)
                   + b1_ref[...])


def _dense1(x_pad, W1l_T, W1r_T, b1):
    return pl.pallas_call(
        _dense1_body,
        grid=(GRID,),
        in_specs=[
            pl.BlockSpec((R, 128), lambda i: (i, 0)),
            pl.BlockSpec((128, 128), lambda i: (0, 0)),
            pl.BlockSpec((128, 128), lambda i: (0, 0)),
            pl.BlockSpec((1, 128), lambda i: (0, 0)),
        ],
        out_specs=[
            pl.BlockSpec((2, R, 64), lambda i: (0, i, 0)),
            pl.BlockSpec((R, 128), lambda i: (i, 0)),
        ],
        out_shape=[
            jax.ShapeDtypeStruct((2, NT, 64), jnp.bfloat16),
            jax.ShapeDtypeStruct((NT, 128), jnp.float32),
        ],
    )(x_pad, W1l_T, W1r_T, b1)


def _dense2_body(agg_ref, cnt_ref, xr_ref, w2l_ref, w2r_ref, b2_ref,
                 hl_ref, hr_ref):
    agg = jnp.concatenate([agg_ref[0], agg_ref[1]], axis=1)
    cnt = cnt_ref[0] + cnt_ref[1]
    inv = 1.0 / jnp.maximum(cnt, 1.0)
    h1 = jnp.maximum(agg * inv[:, None] + xr_ref[...], 0.0)
    hl = jnp.dot(h1, w2l_ref[...], preferred_element_type=jnp.float32)
    hl_ref[...] = hl.astype(jnp.bfloat16)
    hr_ref[...] = (jnp.dot(h1, w2r_ref[...], preferred_element_type=jnp.float32)
                   + b2_ref[...])


def _dense2(aggP, cntP, xr, W2l_T, W2r_T, b2):
    return pl.pallas_call(
        _dense2_body,
        grid=(GRID,),
        in_specs=[
            pl.BlockSpec((2, R, 64), lambda i: (0, i, 0)),
            pl.BlockSpec((2, R), lambda i: (0, i)),
            pl.BlockSpec((R, 128), lambda i: (i, 0)),
            pl.BlockSpec((128, 64), lambda i: (0, 0)),
            pl.BlockSpec((128, 64), lambda i: (0, 0)),
            pl.BlockSpec((1, 64), lambda i: (0, 0)),
        ],
        out_specs=[
            pl.BlockSpec((R, 64), lambda i: (i, 0)),
            pl.BlockSpec((R, 64), lambda i: (i, 0)),
        ],
        out_shape=[
            jax.ShapeDtypeStruct((NT, 64), jnp.bfloat16),
            jax.ShapeDtypeStruct((NT, 64), jnp.float32),
        ],
    )(aggP, cntP, xr, W2l_T, W2r_T, b2)


def _dense3_body(agg_ref, cnt_ref, hr_ref, wfc_ref, bfc_ref, out_ref):
    agg = agg_ref[0] + agg_ref[1]
    cnt = cnt_ref[0] + cnt_ref[1]
    inv = 1.0 / jnp.maximum(cnt, 1.0)
    h2 = jnp.maximum(agg * inv[:, None] + hr_ref[...], 0.0)
    logit = jnp.sum(h2 * wfc_ref[...], axis=1, keepdims=True) + bfc_ref[...]
    out_ref[...] = jax.nn.sigmoid(logit)


def _dense3(agg2P, cntP, hr, Wfc, bfc):
    return pl.pallas_call(
        _dense3_body,
        grid=(GRID,),
        in_specs=[
            pl.BlockSpec((2, R, 64), lambda i: (0, i, 0)),
            pl.BlockSpec((2, R), lambda i: (0, i)),
            pl.BlockSpec((R, 64), lambda i: (i, 0)),
            pl.BlockSpec((1, 64), lambda i: (0, 0)),
            pl.BlockSpec((1, 1), lambda i: (0, 0)),
        ],
        out_specs=pl.BlockSpec((R, 1), lambda i: (i, 0)),
        out_shape=jax.ShapeDtypeStruct((NT, 1), jnp.float32),
    )(agg2P, cntP, hr, Wfc, bfc)


def _make_agg(feature_split):
    n_slabs = NS if feature_split else NW
    nb = EP // (n_slabs * K)
    ngrp = nb // RING

    mesh = plsc.VectorSubcoreMesh(
        core_axis_name="c", subcore_axis_name="s",
        num_cores=NC, num_subcores=NS)

    scratch = [
        pltpu.VMEM((nb, K), jnp.int32),
        pltpu.VMEM((nb, K), jnp.int32),
        [pltpu.VMEM((K, 32), jnp.int32) for _ in range(RING)],
        [pltpu.VMEM((K, 64), jnp.float32) for _ in range(RING)],
        None if feature_split else pltpu.VMEM_SHARED((NT, 32), jnp.int32),
        pltpu.VMEM_SHARED((NT, 64), jnp.float32),
        [pltpu.SemaphoreType.DMA for _ in range(RING)],
        [pltpu.SemaphoreType.DMA for _ in range(RING)],
        pltpu.SemaphoreType.DMA,
    ]
    scratch = [s for s in scratch if s is not None]

    @functools.partial(pl.kernel, mesh=mesh,
                       out_type=jax.ShapeDtypeStruct((NC, NT, 64),
                                                     jnp.float32),
                       scratch_types=scratch,
                       compiler_params=pltpu.CompilerParams(
                           use_tc_tiling_on_sc=False,
                           needs_layout_passes=False))
    def agg_kernel(table, src, dst, zrows, out, *refs):
        if feature_split:
            (srcs, dsts, rowsp, rowsf, acc, gsem, ssem, sem) = refs
        else:
            (srcs, dsts, rowsp, rowsf, stbl, acc, gsem, ssem, sem) = refs

        cid = lax.axis_index("c")
        sid = lax.axis_index("s")
        slab = sid if feature_split else cid * NS + sid

        zbase = sid * ROWS_PER_TILE
        pltpu.async_copy(src.at[slab], srcs, sem)
        pltpu.sync_copy(dst.at[slab], dsts)
        if not feature_split:
            pltpu.sync_copy(table.at[pl.ds(zbase, ROWS_PER_TILE)],
                            stbl.at[pl.ds(zbase, ROWS_PER_TILE)])
        pltpu.sync_copy(zrows, acc.at[pl.ds(zbase, ROWS_PER_TILE)])
        pltpu.make_async_copy(src.at[slab], srcs, sem).wait()
        if feature_split:
            base = cid * NT

            def remap(r, carry):
                for j in range(K // 16):
                    v = srcs[r, pl.ds(16 * j, 16)]
                    srcs[r, pl.ds(16 * j, 16)] = v + base
                return carry
            lax.fori_loop(0, nb, remap, 0)
        plsc.subcore_barrier()

        gsrc = table if feature_split else stbl

        def wait_gather(b):
            pltpu.make_async_copy(gsrc.at[srcs.at[0]], rowsp[b],
                                  gsem[b]).wait()

        def wait_scatter(b):
            pltpu.make_async_copy(rowsf[b], acc.at[dsts.at[0]],
                                  ssem[b]).wait()

        def unpack(b):
            himask = jnp.int32(-65536)

            def cvt(r, carry):
                for half in range(2):
                    v = rowsp[b][r, pl.ds(16 * half, 16)]
                    lo = plsc.bitcast(v << 16, jnp.float32)
                    hi = plsc.bitcast(v & himask, jnp.float32)
                    rowsf[b][r, pl.ds(32 * half, 16)] = lo
                    rowsf[b][r, pl.ds(32 * half + 16, 16)] = hi
                return carry
            lax.fori_loop(0, K, cvt, 0)

        def group(io, first):
            for b in range(RING):
                ib = io * RING + b
                if not first:
                    wait_scatter(b)
                pltpu.async_copy(gsrc.at[srcs.at[ib]], rowsp[b], gsem[b])
            for b in range(RING):
                ib = io * RING + b
                wait_gather(b)
                unpack(b)
                pltpu.async_copy(rowsf[b], acc.at[dsts.at[ib]], ssem[b],
                                 add=True)

        group(0, True)

        def body(io, carry):
            group(io, False)
            return carry

        lax.fori_loop(1, ngrp, body, 0)
        for b in range(RING):
            wait_scatter(b)
        plsc.subcore_barrier()

        pltpu.sync_copy(acc.at[pl.ds(zbase, ROWS_PER_TILE)],
                        out.at[cid, pl.ds(zbase, ROWS_PER_TILE)])

    return agg_kernel


_agg128 = _make_agg(feature_split=True)
_agg64 = _make_agg(feature_split=False)

_NBC = EP // (NW * K)


@functools.partial(
    pl.kernel,
    mesh=plsc.VectorSubcoreMesh(core_axis_name="c", subcore_axis_name="s",
                                num_cores=NC, num_subcores=NS),
    out_type=jax.ShapeDtypeStruct((NC, NT), jnp.float32),
    scratch_types=[
        pltpu.VMEM((_NBC, K), jnp.int32),
        pltpu.VMEM((K,), jnp.float32),
        pltpu.VMEM_SHARED((NT,), jnp.float32),
        pltpu.SemaphoreType.DMA,
        pltpu.SemaphoreType.DMA,
    ],
    compiler_params=pltpu.CompilerParams(use_tc_tiling_on_sc=False,
                                         needs_layout_passes=False))
def _cnt_kernel(dst, zcnt, cnt_out, dsts, ones, cacc, sem, ssem):
    cid = lax.axis_index("c")
    sid = lax.axis_index("s")
    zbase = sid * ROWS_PER_TILE
    pltpu.sync_copy(dst.at[cid * NS + sid], dsts)
    pltpu.sync_copy(zcnt, cacc.at[pl.ds(zbase, ROWS_PER_TILE)])
    for j in range(K // 16):
        ones[pl.ds(16 * j, 16)] = jnp.full((16,), 1.0, jnp.float32)
    plsc.subcore_barrier()

    def body(i, carry):
        pltpu.async_copy(ones, cacc.at[dsts.at[i]], ssem, add=True)
        return carry

    lax.fori_loop(0, _NBC, body, 0)

    def drain(i, carry):
        pltpu.make_async_copy(ones, cacc.at[dsts.at[0]], ssem).wait()
        return carry

    lax.fori_loop(0, _NBC, drain, 0)
    plsc.subcore_barrier()
    pltpu.sync_copy(cacc.at[pl.ds(zbase, ROWS_PER_TILE)],
                    cnt_out.at[cid, pl.ds(zbase, ROWS_PER_TILE)])


def _pack_bf16(t):
    return jax.lax.bitcast_convert_type(
        t.reshape(t.shape[:-1] + (32, 2)), jnp.int32)


def kernel(x, edge_index, W1l, b1, W1r, W2l, b2, W2r, Wfc, bfc):
    x_pad = jnp.pad(x, ((0, NT - N), (0, 0)))
    src_flat = jnp.pad(edge_index[0], (0, EP - E))
    dst_flat = jnp.pad(edge_index[1], (0, EP - E), constant_values=NT - 1)
    src16 = src_flat.reshape(NS, EP // (NS * K), K)
    dst16 = dst_flat.reshape(NS, EP // (NS * K), K)
    src32 = src_flat.reshape(NW, EP // (NW * K), K)
    dst32 = dst_flat.reshape(NW, EP // (NW * K), K)
    zrows64 = jnp.zeros((ROWS_PER_TILE, 64), jnp.float32)
    zcnt = jnp.zeros((ROWS_PER_TILE,), jnp.float32)

    perm128 = np.concatenate([_H, 64 + _H])
    W1lT_p = W1l.T[:, perm128]
    W2lT_p = W2l.T[:, _H]

    cntP = _cnt_kernel(dst32, zcnt)
    xlb, xr = _dense1(x_pad, W1lT_p, W1r.T, b1.reshape(1, 128))
    aggP = _agg128(_pack_bf16(xlb).reshape(2 * NT, 32), src16, dst16,
                   zrows64)
    hl, hr = _dense2(aggP, cntP, xr, W2lT_p, W2r.T, b2.reshape(1, 64))
    agg2P = _agg64(_pack_bf16(hl), src32, dst32, zrows64)
    out = _dense3(agg2P, cntP, hr, Wfc.reshape(1, 64), bfc.reshape(1, 1))
    return out[:N, 0]


# speedup vs baseline: 1.3445x; 1.1172x over previous
import functools

import jax
import jax.numpy as jnp
import numpy as np
from jax import lax
from jax.experimental import pallas as pl
from jax.experimental.pallas import tpu as pltpu
from jax.experimental.pallas import tpu_sc as plsc

N = 10000
E = 320000
NT = 10240
R = 640
GRID = NT // R
NC = 2
NS = 16
NW = NC * NS
K = 128
EP = 327680
RING = 4
ROWS_PER_TILE = NT // NS

_G = np.array([2 * p for p in range(16)] + [2 * p + 1 for p in range(16)]
              + [32 + 2 * p for p in range(16)]
              + [33 + 2 * p for p in range(16)])
_H = np.argsort(_G)


def _dense1_body(x_ref, wl_ref, wr_ref, b1_ref, xlb_ref, xr_ref):
    x = x_ref[...]
    xl = jnp.dot(x, wl_ref[...], preferred_element_type=jnp.float32)
    xlb = xl.astype(jnp.bfloat16)
    xlb_ref[0] = xlb[:, :64]
    xlb_ref[1] = xlb[:, 64:]
    xr_ref[...] = (jnp.dot(x, wr_ref[...], preferred_element_type=jnp.float32)
                   + b1_ref[...])


def _dense1(x_pad, W1l_T, W1r_T, b1):
    return pl.pallas_call(
        _dense1_body,
        grid=(GRID,),
        in_specs=[
            pl.BlockSpec((R, 128), lambda i: (i, 0)),
            pl.BlockSpec((128, 128), lambda i: (0, 0)),
            pl.BlockSpec((128, 128), lambda i: (0, 0)),
            pl.BlockSpec((1, 128), lambda i: (0, 0)),
        ],
        out_specs=[
            pl.BlockSpec((2, R, 64), lambda i: (0, i, 0)),
            pl.BlockSpec((R, 128), lambda i: (i, 0)),
        ],
        out_shape=[
            jax.ShapeDtypeStruct((2, NT, 64), jnp.bfloat16),
            jax.ShapeDtypeStruct((NT, 128), jnp.float32),
        ],
    )(x_pad, W1l_T, W1r_T, b1)


def _dense2_body(agg_ref, cnt_ref, xr_ref, w2l_ref, w2r_ref, b2_ref,
                 hl_ref, hr_ref):
    agg = jnp.concatenate([agg_ref[0, 0] + agg_ref[0, 1],
                           agg_ref[1, 0] + agg_ref[1, 1]], axis=1)
    cnt = cnt_ref[0] + cnt_ref[1]
    inv = 1.0 / jnp.maximum(cnt, 1.0)
    h1 = jnp.maximum(agg * inv[:, None] + xr_ref[...], 0.0)
    hl = jnp.dot(h1, w2l_ref[...], preferred_element_type=jnp.float32)
    hl_ref[...] = hl.astype(jnp.bfloat16)
    hr_ref[...] = (jnp.dot(h1, w2r_ref[...], preferred_element_type=jnp.float32)
                   + b2_ref[...])


def _dense2(aggP, cntP, xr, W2l_T, W2r_T, b2):
    return pl.pallas_call(
        _dense2_body,
        grid=(GRID,),
        in_specs=[
            pl.BlockSpec((2, 2, R, 64), lambda i: (0, 0, i, 0)),
            pl.BlockSpec((2, R), lambda i: (0, i)),
            pl.BlockSpec((R, 128), lambda i: (i, 0)),
            pl.BlockSpec((128, 64), lambda i: (0, 0)),
            pl.BlockSpec((128, 64), lambda i: (0, 0)),
            pl.BlockSpec((1, 64), lambda i: (0, 0)),
        ],
        out_specs=[
            pl.BlockSpec((R, 64), lambda i: (i, 0)),
            pl.BlockSpec((R, 64), lambda i: (i, 0)),
        ],
        out_shape=[
            jax.ShapeDtypeStruct((NT, 64), jnp.bfloat16),
            jax.ShapeDtypeStruct((NT, 64), jnp.float32),
        ],
    )(aggP, cntP, xr, W2l_T, W2r_T, b2)


def _dense3_body(agg_ref, cnt_ref, hr_ref, wfc_ref, bfc_ref, out_ref):
    agg = agg_ref[0] + agg_ref[1]
    cnt = cnt_ref[0] + cnt_ref[1]
    inv = 1.0 / jnp.maximum(cnt, 1.0)
    h2 = jnp.maximum(agg * inv[:, None] + hr_ref[...], 0.0)
    logit = jnp.sum(h2 * wfc_ref[...], axis=1, keepdims=True) + bfc_ref[...]
    out_ref[...] = jax.nn.sigmoid(logit)


def _dense3(agg2P, cntP, hr, Wfc, bfc):
    return pl.pallas_call(
        _dense3_body,
        grid=(GRID,),
        in_specs=[
            pl.BlockSpec((2, R, 64), lambda i: (0, i, 0)),
            pl.BlockSpec((2, R), lambda i: (0, i)),
            pl.BlockSpec((R, 64), lambda i: (i, 0)),
            pl.BlockSpec((1, 64), lambda i: (0, 0)),
            pl.BlockSpec((1, 1), lambda i: (0, 0)),
        ],
        out_specs=pl.BlockSpec((R, 1), lambda i: (i, 0)),
        out_shape=jax.ShapeDtypeStruct((NT, 1), jnp.float32),
    )(agg2P, cntP, hr, Wfc, bfc)


def _make_agg(feature_split):
    nphase = 2 if feature_split else 1
    nb = EP // (NW * K)
    ngrp = nb // RING

    mesh = plsc.VectorSubcoreMesh(
        core_axis_name="c", subcore_axis_name="s",
        num_cores=NC, num_subcores=NS)

    if feature_split:
        out_sds = jax.ShapeDtypeStruct((2, NC, NT, 64), jnp.float32)
    else:
        out_sds = jax.ShapeDtypeStruct((NC, NT, 64), jnp.float32)

    scratch = [
        pltpu.VMEM((nb, K), jnp.int32),
        pltpu.VMEM((nb, K), jnp.int32),
        [pltpu.VMEM((K, 32), jnp.int32) for _ in range(RING)],
        [pltpu.VMEM((K, 64), jnp.float32) for _ in range(RING)],
        pltpu.VMEM_SHARED((NT, 32), jnp.int32),
        pltpu.VMEM_SHARED((NT, 64), jnp.float32),
        [pltpu.SemaphoreType.DMA for _ in range(RING)],
        [pltpu.SemaphoreType.DMA for _ in range(RING)],
        pltpu.SemaphoreType.DMA,
    ]

    @functools.partial(pl.kernel, mesh=mesh,
                       out_type=out_sds,
                       scratch_types=scratch,
                       compiler_params=pltpu.CompilerParams(
                           use_tc_tiling_on_sc=False,
                           needs_layout_passes=False))
    def agg_kernel(table, src, dst, zrows, out,
                   srcs, dsts, rowsp, rowsf, stbl, acc, gsem, ssem, sem):
        cid = lax.axis_index("c")
        sid = lax.axis_index("s")
        slab = cid * NS + sid
        zbase = sid * ROWS_PER_TILE

        pltpu.async_copy(src.at[slab], srcs, sem)
        pltpu.sync_copy(dst.at[slab], dsts)
        pltpu.make_async_copy(src.at[slab], srcs, sem).wait()

        def wait_gather(b):
            pltpu.make_async_copy(stbl.at[srcs.at[0]], rowsp[b],
                                  gsem[b]).wait()

        def wait_scatter(b):
            pltpu.make_async_copy(rowsf[b], acc.at[dsts.at[0]],
                                  ssem[b]).wait()

        def unpack(b):
            himask = jnp.int32(-65536)

            def cvt(r, carry):
                for half in range(2):
                    v = rowsp[b][r, pl.ds(16 * half, 16)]
                    lo = plsc.bitcast(v << 16, jnp.float32)
                    hi = plsc.bitcast(v & himask, jnp.float32)
                    rowsf[b][r, pl.ds(32 * half, 16)] = lo
                    rowsf[b][r, pl.ds(32 * half + 16, 16)] = hi
                return carry
            lax.fori_loop(0, K, cvt, 0)

        def group(io, first):
            for b in range(RING):
                ib = io * RING + b
                if not first:
                    wait_scatter(b)
                pltpu.async_copy(stbl.at[srcs.at[ib]], rowsp[b], gsem[b])
            for b in range(RING):
                ib = io * RING + b
                wait_gather(b)
                unpack(b)
                pltpu.async_copy(rowsf[b], acc.at[dsts.at[ib]], ssem[b],
                                 add=True)

        def body(io, carry):
            group(io, False)
            return carry

        for phase in range(nphase):
            pltpu.sync_copy(
                table.at[pl.ds(phase * NT + zbase, ROWS_PER_TILE)]
                if feature_split else table.at[pl.ds(zbase, ROWS_PER_TILE)],
                stbl.at[pl.ds(zbase, ROWS_PER_TILE)])
            pltpu.sync_copy(zrows, acc.at[pl.ds(zbase, ROWS_PER_TILE)])
            plsc.subcore_barrier()

            group(0, True)
            lax.fori_loop(1, ngrp, body, 0)
            for b in range(RING):
                wait_scatter(b)
            plsc.subcore_barrier()

            dst_ref = (out.at[phase, cid] if feature_split
                       else out.at[cid])
            pltpu.sync_copy(acc.at[pl.ds(zbase, ROWS_PER_TILE)],
                            dst_ref.at[pl.ds(zbase, ROWS_PER_TILE)])

    return agg_kernel


_agg128 = _make_agg(feature_split=True)
_agg64 = _make_agg(feature_split=False)

_NBC = EP // (NW * K)


@functools.partial(
    pl.kernel,
    mesh=plsc.VectorSubcoreMesh(core_axis_name="c", subcore_axis_name="s",
                                num_cores=NC, num_subcores=NS),
    out_type=jax.ShapeDtypeStruct((NC, NT), jnp.float32),
    scratch_types=[
        pltpu.VMEM((_NBC, K), jnp.int32),
        pltpu.VMEM((K,), jnp.float32),
        pltpu.VMEM_SHARED((NT,), jnp.float32),
        pltpu.SemaphoreType.DMA,
        pltpu.SemaphoreType.DMA,
    ],
    compiler_params=pltpu.CompilerParams(use_tc_tiling_on_sc=False,
                                         needs_layout_passes=False))
def _cnt_kernel(dst, zcnt, cnt_out, dsts, ones, cacc, sem, ssem):
    cid = lax.axis_index("c")
    sid = lax.axis_index("s")
    zbase = sid * ROWS_PER_TILE
    pltpu.sync_copy(dst.at[cid * NS + sid], dsts)
    pltpu.sync_copy(zcnt, cacc.at[pl.ds(zbase, ROWS_PER_TILE)])
    for j in range(K // 16):
        ones[pl.ds(16 * j, 16)] = jnp.full((16,), 1.0, jnp.float32)
    plsc.subcore_barrier()

    def body(i, carry):
        pltpu.async_copy(ones, cacc.at[dsts.at[i]], ssem, add=True)
        return carry

    lax.fori_loop(0, _NBC, body, 0)

    def drain(i, carry):
        pltpu.make_async_copy(ones, cacc.at[dsts.at[0]], ssem).wait()
        return carry

    lax.fori_loop(0, _NBC, drain, 0)
    plsc.subcore_barrier()
    pltpu.sync_copy(cacc.at[pl.ds(zbase, ROWS_PER_TILE)],
                    cnt_out.at[cid, pl.ds(zbase, ROWS_PER_TILE)])


def _pack_bf16(t):
    return jax.lax.bitcast_convert_type(
        t.reshape(t.shape[:-1] + (32, 2)), jnp.int32)


def kernel(x, edge_index, W1l, b1, W1r, W2l, b2, W2r, Wfc, bfc):
    x_pad = jnp.pad(x, ((0, NT - N), (0, 0)))
    src_flat = jnp.pad(edge_index[0], (0, EP - E))
    dst_flat = jnp.pad(edge_index[1], (0, EP - E), constant_values=NT - 1)
    src32 = src_flat.reshape(NW, EP // (NW * K), K)
    dst32 = dst_flat.reshape(NW, EP // (NW * K), K)
    zrows64 = jnp.zeros((ROWS_PER_TILE, 64), jnp.float32)
    zcnt = jnp.zeros((ROWS_PER_TILE,), jnp.float32)

    perm128 = np.concatenate([_H, 64 + _H])
    W1lT_p = W1l.T[:, perm128]
    W2lT_p = W2l.T[:, _H]

    cntP = _cnt_kernel(dst32, zcnt)
    xlb, xr = _dense1(x_pad, W1lT_p, W1r.T, b1.reshape(1, 128))
    aggP = _agg128(_pack_bf16(xlb).reshape(2 * NT, 32), src32, dst32,
                   zrows64)
    hl, hr = _dense2(aggP, cntP, xr, W2lT_p, W2r.T, b2.reshape(1, 64))
    agg2P = _agg64(_pack_bf16(hl), src32, dst32, zrows64)
    out = _dense3(agg2P, cntP, hr, Wfc.reshape(1, 64), bfc.reshape(1, 1))
    return out[:N, 0]


# speedup vs baseline: 1.3539x; 1.0070x over previous
import functools

import jax
import jax.numpy as jnp
import numpy as np
from jax import lax
from jax.experimental import pallas as pl
from jax.experimental.pallas import tpu as pltpu
from jax.experimental.pallas import tpu_sc as plsc

N = 10000
E = 320000
NT = 10240
R = 640
GRID = NT // R
NC = 2
NS = 16
NW = NC * NS
K = 128
EP = 327680
RING = 4
ROWS_PER_TILE = NT // NS

_G = np.array([2 * p for p in range(16)] + [2 * p + 1 for p in range(16)]
              + [32 + 2 * p for p in range(16)]
              + [33 + 2 * p for p in range(16)])
_H = np.argsort(_G)


def _dense1_body(x_ref, wl_ref, wr_ref, b1_ref, xlb_ref, xr_ref):
    x = x_ref[...]
    xl = jnp.dot(x, wl_ref[...], preferred_element_type=jnp.float32)
    xlb = xl.astype(jnp.bfloat16)
    xlb_ref[0] = xlb[:, :64]
    xlb_ref[1] = xlb[:, 64:]
    xr_ref[...] = (jnp.dot(x, wr_ref[...], preferred_element_type=jnp.float32)
                   + b1_ref[...])


def _dense1(x_pad, W1l_T, W1r_T, b1):
    return pl.pallas_call(
        _dense1_body,
        grid=(GRID,),
        in_specs=[
            pl.BlockSpec((R, 128), lambda i: (i, 0)),
            pl.BlockSpec((128, 128), lambda i: (0, 0)),
            pl.BlockSpec((128, 128), lambda i: (0, 0)),
            pl.BlockSpec((1, 128), lambda i: (0, 0)),
        ],
        out_specs=[
            pl.BlockSpec((2, R, 64), lambda i: (0, i, 0)),
            pl.BlockSpec((R, 128), lambda i: (i, 0)),
        ],
        out_shape=[
            jax.ShapeDtypeStruct((2, NT, 64), jnp.bfloat16),
            jax.ShapeDtypeStruct((NT, 128), jnp.float32),
        ],
    )(x_pad, W1l_T, W1r_T, b1)


def _h1_of(agg_ref, cnt_ref, xr_ref):
    agg = jnp.concatenate([agg_ref[0, 0] + agg_ref[0, 1],
                           agg_ref[1, 0] + agg_ref[1, 1]], axis=1)
    cnt = cnt_ref[0] + cnt_ref[1]
    inv = 1.0 / jnp.maximum(cnt, 1.0)
    return jnp.maximum(agg * inv[:, None] + xr_ref[...], 0.0)


def _dense2a_body(agg_ref, cnt_ref, xr_ref, w2l_ref, hl_ref):
    h1 = _h1_of(agg_ref, cnt_ref, xr_ref)
    hl = jnp.dot(h1, w2l_ref[...], preferred_element_type=jnp.float32)
    hl_ref[...] = hl.astype(jnp.bfloat16)


def _dense2b_body(agg_ref, cnt_ref, xr_ref, w2r_ref, b2_ref, hr_ref):
    h1 = _h1_of(agg_ref, cnt_ref, xr_ref)
    hr_ref[...] = (jnp.dot(h1, w2r_ref[...], preferred_element_type=jnp.float32)
                   + b2_ref[...])


_D2_SPECS = [
    pl.BlockSpec((2, 2, R, 64), lambda i: (0, 0, i, 0)),
    pl.BlockSpec((2, R), lambda i: (0, i)),
    pl.BlockSpec((R, 128), lambda i: (i, 0)),
    pl.BlockSpec((128, 64), lambda i: (0, 0)),
]


def _dense2a(aggP, cntP, xr, W2l_T):
    return pl.pallas_call(
        _dense2a_body,
        grid=(GRID,),
        in_specs=_D2_SPECS,
        out_specs=pl.BlockSpec((R, 64), lambda i: (i, 0)),
        out_shape=jax.ShapeDtypeStruct((NT, 64), jnp.bfloat16),
    )(aggP, cntP, xr, W2l_T)


def _dense2b(aggP, cntP, xr, W2r_T, b2):
    return pl.pallas_call(
        _dense2b_body,
        grid=(GRID,),
        in_specs=_D2_SPECS + [pl.BlockSpec((1, 64), lambda i: (0, 0))],
        out_specs=pl.BlockSpec((R, 64), lambda i: (i, 0)),
        out_shape=jax.ShapeDtypeStruct((NT, 64), jnp.float32),
    )(aggP, cntP, xr, W2r_T, b2)


def _dense3_body(agg_ref, cnt_ref, hr_ref, wfc_ref, bfc_ref, out_ref):
    agg = agg_ref[0] + agg_ref[1]
    cnt = cnt_ref[0] + cnt_ref[1]
    inv = 1.0 / jnp.maximum(cnt, 1.0)
    h2 = jnp.maximum(agg * inv[:, None] + hr_ref[...], 0.0)
    logit = jnp.sum(h2 * wfc_ref[...], axis=1, keepdims=True) + bfc_ref[...]
    out_ref[...] = jax.nn.sigmoid(logit)


def _dense3(agg2P, cntP, hr, Wfc, bfc):
    return pl.pallas_call(
        _dense3_body,
        grid=(GRID,),
        in_specs=[
            pl.BlockSpec((2, R, 64), lambda i: (0, i, 0)),
            pl.BlockSpec((2, R), lambda i: (0, i)),
            pl.BlockSpec((R, 64), lambda i: (i, 0)),
            pl.BlockSpec((1, 64), lambda i: (0, 0)),
            pl.BlockSpec((1, 1), lambda i: (0, 0)),
        ],
        out_specs=pl.BlockSpec((R, 1), lambda i: (i, 0)),
        out_shape=jax.ShapeDtypeStruct((NT, 1), jnp.float32),
    )(agg2P, cntP, hr, Wfc, bfc)


def _make_agg(feature_split):
    nphase = 2 if feature_split else 1
    nb = EP // (NW * K)
    ngrp = nb // RING

    mesh = plsc.VectorSubcoreMesh(
        core_axis_name="c", subcore_axis_name="s",
        num_cores=NC, num_subcores=NS)

    if feature_split:
        out_sds = jax.ShapeDtypeStruct((2, NC, NT, 64), jnp.float32)
    else:
        out_sds = jax.ShapeDtypeStruct((NC, NT, 64), jnp.float32)

    scratch = [
        pltpu.VMEM((nb, K), jnp.int32),
        pltpu.VMEM((nb, K), jnp.int32),
        [pltpu.VMEM((K, 32), jnp.int32) for _ in range(RING)],
        [pltpu.VMEM((K, 64), jnp.float32) for _ in range(RING)],
        pltpu.VMEM_SHARED((NT, 32), jnp.int32),
        pltpu.VMEM_SHARED((NT, 64), jnp.float32),
        [pltpu.SemaphoreType.DMA for _ in range(RING)],
        [pltpu.SemaphoreType.DMA for _ in range(RING)],
        pltpu.SemaphoreType.DMA,
    ]

    @functools.partial(pl.kernel, mesh=mesh,
                       out_type=out_sds,
                       scratch_types=scratch,
                       compiler_params=pltpu.CompilerParams(
                           use_tc_tiling_on_sc=False,
                           needs_layout_passes=False))
    def agg_kernel(table, src, dst, zrows, out,
                   srcs, dsts, rowsp, rowsf, stbl, acc, gsem, ssem, sem):
        cid = lax.axis_index("c")
        sid = lax.axis_index("s")
        slab = cid * NS + sid
        zbase = sid * ROWS_PER_TILE

        pltpu.async_copy(src.at[slab], srcs, sem)
        pltpu.sync_copy(dst.at[slab], dsts)
        pltpu.make_async_copy(src.at[slab], srcs, sem).wait()

        def wait_gather(b):
            pltpu.make_async_copy(stbl.at[srcs.at[0]], rowsp[b],
                                  gsem[b]).wait()

        def wait_scatter(b):
            pltpu.make_async_copy(rowsf[b], acc.at[dsts.at[0]],
                                  ssem[b]).wait()

        def unpack(b):
            himask = jnp.int32(-65536)

            def cvt(r, carry):
                for half in range(2):
                    v = rowsp[b][r, pl.ds(16 * half, 16)]
                    lo = plsc.bitcast(v << 16, jnp.float32)
                    hi = plsc.bitcast(v & himask, jnp.float32)
                    rowsf[b][r, pl.ds(32 * half, 16)] = lo
                    rowsf[b][r, pl.ds(32 * half + 16, 16)] = hi
                return carry
            lax.fori_loop(0, K, cvt, 0)

        def group(io, first):
            for b in range(RING):
                ib = io * RING + b
                if not first:
                    wait_scatter(b)
                pltpu.async_copy(stbl.at[srcs.at[ib]], rowsp[b], gsem[b])
            for b in range(RING):
                ib = io * RING + b
                wait_gather(b)
                unpack(b)
                pltpu.async_copy(rowsf[b], acc.at[dsts.at[ib]], ssem[b],
                                 add=True)

        def body(io, carry):
            group(io, False)
            return carry

        for phase in range(nphase):
            pltpu.sync_copy(
                table.at[pl.ds(phase * NT + zbase, ROWS_PER_TILE)]
                if feature_split else table.at[pl.ds(zbase, ROWS_PER_TILE)],
                stbl.at[pl.ds(zbase, ROWS_PER_TILE)])
            pltpu.sync_copy(zrows, acc.at[pl.ds(zbase, ROWS_PER_TILE)])
            plsc.subcore_barrier()

            group(0, True)
            lax.fori_loop(1, ngrp, body, 0)
            for b in range(RING):
                wait_scatter(b)
            plsc.subcore_barrier()

            dst_ref = (out.at[phase, cid] if feature_split
                       else out.at[cid])
            pltpu.sync_copy(acc.at[pl.ds(zbase, ROWS_PER_TILE)],
                            dst_ref.at[pl.ds(zbase, ROWS_PER_TILE)])

    return agg_kernel


_agg128 = _make_agg(feature_split=True)
_agg64 = _make_agg(feature_split=False)

_NBC = EP // (NW * K)


@functools.partial(
    pl.kernel,
    mesh=plsc.VectorSubcoreMesh(core_axis_name="c", subcore_axis_name="s",
                                num_cores=NC, num_subcores=NS),
    out_type=jax.ShapeDtypeStruct((NC, NT), jnp.float32),
    scratch_types=[
        pltpu.VMEM((_NBC, K), jnp.int32),
        pltpu.VMEM((K,), jnp.float32),
        pltpu.VMEM_SHARED((NT,), jnp.float32),
        pltpu.SemaphoreType.DMA,
        pltpu.SemaphoreType.DMA,
    ],
    compiler_params=pltpu.CompilerParams(use_tc_tiling_on_sc=False,
                                         needs_layout_passes=False))
def _cnt_kernel(dst, zcnt, cnt_out, dsts, ones, cacc, sem, ssem):
    cid = lax.axis_index("c")
    sid = lax.axis_index("s")
    zbase = sid * ROWS_PER_TILE
    pltpu.sync_copy(dst.at[cid * NS + sid], dsts)
    pltpu.sync_copy(zcnt, cacc.at[pl.ds(zbase, ROWS_PER_TILE)])
    for j in range(K // 16):
        ones[pl.ds(16 * j, 16)] = jnp.full((16,), 1.0, jnp.float32)
    plsc.subcore_barrier()

    def body(i, carry):
        pltpu.async_copy(ones, cacc.at[dsts.at[i]], ssem, add=True)
        return carry

    lax.fori_loop(0, _NBC, body, 0)

    def drain(i, carry):
        pltpu.make_async_copy(ones, cacc.at[dsts.at[0]], ssem).wait()
        return carry

    lax.fori_loop(0, _NBC, drain, 0)
    plsc.subcore_barrier()
    pltpu.sync_copy(cacc.at[pl.ds(zbase, ROWS_PER_TILE)],
                    cnt_out.at[cid, pl.ds(zbase, ROWS_PER_TILE)])


def _pack_bf16(t):
    return jax.lax.bitcast_convert_type(
        t.reshape(t.shape[:-1] + (32, 2)), jnp.int32)


def kernel(x, edge_index, W1l, b1, W1r, W2l, b2, W2r, Wfc, bfc):
    x_pad = jnp.pad(x, ((0, NT - N), (0, 0)))
    src_flat = jnp.pad(edge_index[0], (0, EP - E))
    dst_flat = jnp.pad(edge_index[1], (0, EP - E), constant_values=NT - 1)
    src32 = src_flat.reshape(NW, EP // (NW * K), K)
    dst32 = dst_flat.reshape(NW, EP // (NW * K), K)
    zrows64 = jnp.zeros((ROWS_PER_TILE, 64), jnp.float32)
    zcnt = jnp.zeros((ROWS_PER_TILE,), jnp.float32)

    perm128 = np.concatenate([_H, 64 + _H])
    W1lT_p = W1l.T[:, perm128]
    W2lT_p = W2l.T[:, _H]

    cntP = _cnt_kernel(dst32, zcnt)
    xlb, xr = _dense1(x_pad, W1lT_p, W1r.T, b1.reshape(1, 128))
    aggP = _agg128(_pack_bf16(xlb).reshape(2 * NT, 32), src32, dst32,
                   zrows64)
    hl = _dense2a(aggP, cntP, xr, W2lT_p)
    agg2P = _agg64(_pack_bf16(hl), src32, dst32, zrows64)
    hr = _dense2b(aggP, cntP, xr, W2r.T, b2.reshape(1, 64))
    out = _dense3(agg2P, cntP, hr, Wfc.reshape(1, 64), bfc.reshape(1, 1))
    return out[:N, 0]
